# baseline (speedup 1.0000x reference)
import jax
import jax.numpy as jnp
from jax import lax
from jax.experimental import pallas as pl
from jax.experimental.pallas import tpu as pltpu

_N_EMBED = 8192
_DIM = 32
_M = 512
_CK = 512
_NCHUNK = _N_EMBED // _CK
_BETA = 0.25
_EPS = 1e-5


def _stage_body(z_ref, e_ref, z2_ref, e2_ref,
                idx_ref, zq_ref, loss_ref,
                counts_ref, sq_ref):
    pid = pl.program_id(0)
    nsteps = pl.num_programs(0)

    @pl.when(pid == 0)
    def _init():
        counts_ref[...] = jnp.zeros_like(counts_ref)
        sq_ref[0] = 0.0

    zt = z_ref[...]
    z2 = z2_ref[...]

    def dbody(c, carry):
        bestv, besti = carry
        off = pl.multiple_of(c * _CK, _CK)
        ech = e_ref[pl.ds(off, _CK), :]
        e2c = e2_ref[pl.ds(off, _CK)]
        mm = lax.dot_general(zt, ech, (((1,), (1,)), ((), ())),
                             preferred_element_type=jnp.float32)
        d = (z2 + e2c[None, :]) - 2.0 * mm
        lv = jnp.min(d, axis=1)
        iota = lax.broadcasted_iota(jnp.int32, (_M, _CK), 1)
        li = jnp.min(jnp.where(d == lv[:, None], iota, _CK), axis=1) + off
        upd = lv < bestv
        return jnp.where(upd, lv, bestv), jnp.where(upd, li, besti)

    bestv0 = jnp.full((_M,), jnp.inf, dtype=jnp.float32)
    besti0 = jnp.zeros((_M,), dtype=jnp.int32)
    _, idx = lax.fori_loop(0, _NCHUNK, dbody, (bestv0, besti0))
    idx_ref[...] = idx

    def gbody(c, acc):
        off = pl.multiple_of(c * _CK, _CK)
        ech = e_ref[pl.ds(off, _CK), :]
        iota = lax.broadcasted_iota(jnp.int32, (_M, _CK), 1) + off
        oh = (idx[:, None] == iota).astype(jnp.float32)
        counts_ref[pl.ds(c, 1), :] += jnp.sum(oh, axis=0)[None, :]
        return acc + lax.dot_general(oh, ech, (((1,), (0,)), ((), ())),
                                     preferred_element_type=jnp.float32)

    zq = lax.fori_loop(0, _NCHUNK, gbody,
                       jnp.zeros((_M, _DIM), dtype=jnp.float32))
    zq_ref[...] = zq
    sq_ref[0] += jnp.sum((zq - zt) ** 2)

    @pl.when(pid == nsteps - 1)
    def _fin():
        numel = jnp.asarray(nsteps * _M * _DIM, dtype=jnp.float32)
        counts = counts_ref[...]
        tot = jnp.sum(counts)
        probs = (counts + _EPS) / (tot + _EPS * _N_EMBED)
        ent = -jnp.sum(probs * jnp.log(probs + _EPS))
        usage = jnp.log(jnp.float32(_N_EMBED)) - ent
        loss = _BETA * sq_ref[0] / numel + 0.01 * usage
        loss_ref[...] = jnp.broadcast_to(loss, (1, 1))


def _quantize_stage(zf, emb, z2, e2):
    n_pix = zf.shape[0]
    grid = n_pix // _M
    return pl.pallas_call(
        _stage_body,
        grid=(grid,),
        in_specs=[
            pl.BlockSpec((_M, _DIM), lambda i: (i, 0)),
            pl.BlockSpec((_N_EMBED, _DIM), lambda i: (0, 0)),
            pl.BlockSpec((_M, 1), lambda i: (i, 0)),
            pl.BlockSpec((_N_EMBED,), lambda i: (0,)),
        ],
        out_specs=[
            pl.BlockSpec((_M,), lambda i: (i,)),
            pl.BlockSpec((_M, _DIM), lambda i: (i, 0)),
            pl.BlockSpec((1, 1), lambda i: (0, 0)),
        ],
        out_shape=[
            jax.ShapeDtypeStruct((n_pix,), jnp.int32),
            jax.ShapeDtypeStruct((n_pix, _DIM), jnp.float32),
            jax.ShapeDtypeStruct((1, 1), jnp.float32),
        ],
        scratch_shapes=[
            pltpu.VMEM((_NCHUNK, _CK), jnp.float32),
            pltpu.SMEM((1,), jnp.float32),
        ],
    )(zf, emb, z2, e2)


def kernel(z, emb0, emb1):
    B, C, H, W = z.shape
    assert B == 1 and C == _DIM
    h, w = H // 4, W // 4
    zf = z.reshape(C, H * W).T

    e2_0 = jnp.sum(emb0 ** 2, axis=1)
    e2_1 = jnp.sum(emb1 ** 2, axis=1)

    z2_0 = jnp.sum(zf ** 2, axis=1, keepdims=True)
    i0, zq0, loss0 = _quantize_stage(zf, emb0, z2_0, e2_0)

    resid = zf - zq0
    z2_1 = jnp.sum(resid ** 2, axis=1, keepdims=True)
    i1, zq1, loss1 = _quantize_stage(resid, emb1, z2_1, e2_1)

    z_q = (zq0 + zq1).T.reshape(B, C, H, W)

    def to_mei(ix):
        return ix.reshape(h, 4, w, 4).transpose(0, 2, 1, 3).reshape(1, h, w, 16)

    mei = jnp.stack([to_mei(i0), to_mei(i1)], axis=-1)
    return z_q, loss0[0, 0] + loss1[0, 0], mei


# speedup vs baseline: 1.2063x; 1.2063x over previous
import functools

import jax
import jax.numpy as jnp
from jax import lax
from jax.experimental import pallas as pl
from jax.experimental.pallas import tpu as pltpu
from jax.experimental.pallas import tpu_sc as plsc

_N_EMBED = 8192
_DIM = 32
_M = 512
_CK = 512
_NCHUNK = _N_EMBED // _CK
_BETA = 0.25
_EPS = 1e-5


def _stage_body(z_ref, e_ref, z2_ref, e2_ref,
                idx_ref, loss_ref, counts_ref, sq_ref):
    pid = pl.program_id(0)
    nsteps = pl.num_programs(0)

    @pl.when(pid == 0)
    def _init():
        counts_ref[...] = jnp.zeros_like(counts_ref)
        sq_ref[0] = 0.0

    zt = z_ref[...]
    z2 = z2_ref[...]

    def dbody(c, carry):
        bestv, besti = carry
        off = pl.multiple_of(c * _CK, _CK)
        ech = e_ref[pl.ds(off, _CK), :]
        e2c = e2_ref[pl.ds(off, _CK)]
        mm = lax.dot_general(zt, ech, (((1,), (1,)), ((), ())),
                             preferred_element_type=jnp.float32)
        d = (z2 + e2c[None, :]) - 2.0 * mm
        lv = jnp.min(d, axis=1)
        iota = lax.broadcasted_iota(jnp.int32, (_M, _CK), 1)
        li = jnp.min(jnp.where(d == lv[:, None], iota, _CK), axis=1) + off
        upd = lv < bestv
        return jnp.where(upd, lv, bestv), jnp.where(upd, li, besti)

    bestv0 = jnp.full((_M,), jnp.inf, dtype=jnp.float32)
    besti0 = jnp.zeros((_M,), dtype=jnp.int32)
    bestv, idx = lax.fori_loop(0, _NCHUNK, dbody, (bestv0, besti0))
    idx_ref[...] = idx
    sq_ref[0] += jnp.sum(jnp.maximum(bestv, 0.0))

    def cbody(c, _):
        off = pl.multiple_of(c * _CK, _CK)
        iota = lax.broadcasted_iota(jnp.int32, (_M, _CK), 1) + off
        oh = (idx[:, None] == iota).astype(jnp.float32)
        counts_ref[pl.ds(c, 1), :] += jnp.sum(oh, axis=0)[None, :]
        return 0

    lax.fori_loop(0, _NCHUNK, cbody, 0)

    @pl.when(pid == nsteps - 1)
    def _fin():
        numel = jnp.asarray(nsteps * _M * _DIM, dtype=jnp.float32)
        counts = counts_ref[...]
        tot = jnp.sum(counts)
        probs = (counts + _EPS) / (tot + _EPS * _N_EMBED)
        ent = -jnp.sum(probs * jnp.log(probs + _EPS))
        usage = jnp.log(jnp.float32(_N_EMBED)) - ent
        loss = _BETA * sq_ref[0] / numel + 0.01 * usage
        loss_ref[...] = jnp.broadcast_to(loss, (1, 1))


def _quantize_stage(zf, emb, z2, e2):
    n_pix = zf.shape[0]
    grid = n_pix // _M
    return pl.pallas_call(
        _stage_body,
        grid=(grid,),
        in_specs=[
            pl.BlockSpec((_M, _DIM), lambda i: (i, 0)),
            pl.BlockSpec((_N_EMBED, _DIM), lambda i: (0, 0)),
            pl.BlockSpec((_M, 1), lambda i: (i, 0)),
            pl.BlockSpec((_N_EMBED,), lambda i: (0,)),
        ],
        out_specs=[
            pl.BlockSpec((_M,), lambda i: (i,)),
            pl.BlockSpec((1, 1), lambda i: (0, 0)),
        ],
        out_shape=[
            jax.ShapeDtypeStruct((n_pix,), jnp.int32),
            jax.ShapeDtypeStruct((1, 1), jnp.float32),
        ],
        scratch_shapes=[
            pltpu.VMEM((_NCHUNK, _CK), jnp.float32),
            pltpu.SMEM((1,), jnp.float32),
        ],
    )(zf, emb, z2, e2)


_GW = 128


def _sc_gather(table128, idx):
    n = idx.shape[0]
    info = plsc.get_sparse_core_info()
    nw = info.num_cores * info.num_subcores
    b_per_w = n // nw
    nchunk = 2
    bc = b_per_w // nchunk
    mesh = plsc.VectorSubcoreMesh(core_axis_name="c", subcore_axis_name="s")

    @functools.partial(
        pl.kernel, mesh=mesh,
        out_type=jax.ShapeDtypeStruct((n, _GW), jnp.float32),
        scratch_types=[
            pltpu.VMEM((bc,), jnp.int32),
            pltpu.VMEM((bc, _GW), jnp.float32),
            pltpu.SemaphoreType.DMA,
        ],
    )
    def k(table_hbm, idx_hbm, out_hbm, idx_v, rows_v, sem):
        wid = lax.axis_index("s") * info.num_cores + lax.axis_index("c")
        for j in range(nchunk):
            base = wid * b_per_w + j * bc
            pltpu.sync_copy(idx_hbm.at[pl.ds(base, bc)], idx_v)
            pltpu.async_copy(table_hbm.at[idx_v], rows_v, sem).wait()
            pltpu.sync_copy(rows_v, out_hbm.at[pl.ds(base, bc)])

    return k(table128, idx)


def kernel(z, emb0, emb1):
    B, C, H, W = z.shape
    assert B == 1 and C == _DIM
    h, w = H // 4, W // 4
    zf = z.reshape(C, H * W).T

    e2_0 = jnp.sum(emb0 ** 2, axis=1)
    e2_1 = jnp.sum(emb1 ** 2, axis=1)
    pad = ((0, 0), (0, _GW - _DIM))
    emb0p = jnp.pad(emb0, pad)
    emb1p = jnp.pad(emb1, pad)

    z2_0 = jnp.sum(zf ** 2, axis=1, keepdims=True)
    i0, loss0 = _quantize_stage(zf, emb0, z2_0, e2_0)
    zq0 = _sc_gather(emb0p, i0)[:, :_DIM]

    resid = zf - zq0
    z2_1 = jnp.sum(resid ** 2, axis=1, keepdims=True)
    i1, loss1 = _quantize_stage(resid, emb1, z2_1, e2_1)
    zq1 = _sc_gather(emb1p, i1)[:, :_DIM]

    z_q = (zq0 + zq1).T.reshape(B, C, H, W)

    def to_mei(ix):
        return ix.reshape(h, 4, w, 4).transpose(0, 2, 1, 3).reshape(1, h, w, 16)

    mei = jnp.stack([to_mei(i0), to_mei(i1)], axis=-1)
    return z_q, loss0[0, 0] + loss1[0, 0], mei


# speedup vs baseline: 1.3866x; 1.1495x over previous
import functools

import jax
import jax.numpy as jnp
from jax import lax
from jax.experimental import pallas as pl
from jax.experimental.pallas import tpu as pltpu
from jax.experimental.pallas import tpu_sc as plsc

_N_EMBED = 8192
_DIM = 32
_M = 1024
_CK = 512
_NCHUNK = _N_EMBED // _CK
_BETA = 0.25
_EPS = 1e-5


def _stage_body(z_ref, e_ref, z2_ref, e2_ref,
                idx_ref, loss_ref, counts_ref, sq_ref):
    pid = pl.program_id(0)
    nsteps = pl.num_programs(0)

    @pl.when(pid == 0)
    def _init():
        counts_ref[...] = jnp.zeros_like(counts_ref)
        sq_ref[0] = 0.0

    zt = z_ref[...]
    z2 = z2_ref[...]

    def dbody(c, carry):
        bestv, besti = carry
        off = pl.multiple_of(c * _CK, _CK)
        ech = e_ref[pl.ds(off, _CK), :]
        e2c = e2_ref[pl.ds(off, _CK)]
        mm = lax.dot_general(zt, ech, (((1,), (1,)), ((), ())),
                             preferred_element_type=jnp.float32)
        d = (z2 + e2c[None, :]) - 2.0 * mm
        lv = jnp.min(d, axis=1)
        iota = lax.broadcasted_iota(jnp.int32, (_M, _CK), 1)
        li = jnp.min(jnp.where(d == lv[:, None], iota, _CK), axis=1) + off
        upd = lv < bestv
        return jnp.where(upd, lv, bestv), jnp.where(upd, li, besti)

    bestv0 = jnp.full((_M,), jnp.inf, dtype=jnp.float32)
    besti0 = jnp.zeros((_M,), dtype=jnp.int32)
    bestv, idx = lax.fori_loop(0, _NCHUNK, dbody, (bestv0, besti0))
    idx_ref[...] = idx
    sq_ref[0] += jnp.sum(jnp.maximum(bestv, 0.0))

    def cbody(c, _):
        off = pl.multiple_of(c * _CK, _CK)
        iota = lax.broadcasted_iota(jnp.int32, (_M, _CK), 1) + off
        oh = (idx[:, None] == iota).astype(jnp.float32)
        counts_ref[pl.ds(c, 1), :] += jnp.sum(oh, axis=0)[None, :]
        return 0

    lax.fori_loop(0, _NCHUNK, cbody, 0)

    @pl.when(pid == nsteps - 1)
    def _fin():
        numel = jnp.asarray(nsteps * _M * _DIM, dtype=jnp.float32)
        counts = counts_ref[...]
        tot = jnp.sum(counts)
        probs = (counts + _EPS) / (tot + _EPS * _N_EMBED)
        ent = -jnp.sum(probs * jnp.log(probs + _EPS))
        usage = jnp.log(jnp.float32(_N_EMBED)) - ent
        loss = _BETA * sq_ref[0] / numel + 0.01 * usage
        loss_ref[...] = jnp.broadcast_to(loss, (1, 1))


def _quantize_stage(zf, emb, z2, e2):
    n_pix = zf.shape[0]
    grid = n_pix // _M
    return pl.pallas_call(
        _stage_body,
        grid=(grid,),
        in_specs=[
            pl.BlockSpec((_M, _DIM), lambda i: (i, 0)),
            pl.BlockSpec((_N_EMBED, _DIM), lambda i: (0, 0)),
            pl.BlockSpec((_M, 1), lambda i: (i, 0)),
            pl.BlockSpec((_N_EMBED,), lambda i: (0,)),
        ],
        out_specs=[
            pl.BlockSpec((_M,), lambda i: (i,)),
            pl.BlockSpec((1, 1), lambda i: (0, 0)),
        ],
        out_shape=[
            jax.ShapeDtypeStruct((n_pix,), jnp.int32),
            jax.ShapeDtypeStruct((1, 1), jnp.float32),
        ],
        scratch_shapes=[
            pltpu.VMEM((_NCHUNK, _CK), jnp.float32),
            pltpu.SMEM((1,), jnp.float32),
        ],
    )(zf, emb, z2, e2)


_GW = 128


def _sc_gather(table128, idx):
    n = idx.shape[0]
    info = plsc.get_sparse_core_info()
    nw = info.num_cores * info.num_subcores
    b_per_w = n // nw
    nchunk = 2
    bc = b_per_w // nchunk
    mesh = plsc.VectorSubcoreMesh(core_axis_name="c", subcore_axis_name="s")

    @functools.partial(
        pl.kernel, mesh=mesh,
        out_type=jax.ShapeDtypeStruct((n, _GW), jnp.float32),
        scratch_types=[
            pltpu.VMEM((bc,), jnp.int32),
            pltpu.VMEM((bc, _GW), jnp.float32),
            pltpu.SemaphoreType.DMA,
        ],
    )
    def k(table_hbm, idx_hbm, out_hbm, idx_v, rows_v, sem):
        wid = lax.axis_index("s") * info.num_cores + lax.axis_index("c")
        for j in range(nchunk):
            base = wid * b_per_w + j * bc
            pltpu.sync_copy(idx_hbm.at[pl.ds(base, bc)], idx_v)
            pltpu.async_copy(table_hbm.at[idx_v], rows_v, sem).wait()
            pltpu.sync_copy(rows_v, out_hbm.at[pl.ds(base, bc)])

    return k(table128, idx)


def kernel(z, emb0, emb1):
    B, C, H, W = z.shape
    assert B == 1 and C == _DIM
    h, w = H // 4, W // 4
    zf = z.reshape(C, H * W).T

    e2_0 = jnp.sum(emb0 ** 2, axis=1)
    e2_1 = jnp.sum(emb1 ** 2, axis=1)
    pad = ((0, 0), (0, _GW - _DIM))
    emb0p = jnp.pad(emb0, pad)
    emb1p = jnp.pad(emb1, pad)

    z2_0 = jnp.sum(zf ** 2, axis=1, keepdims=True)
    i0, loss0 = _quantize_stage(zf, emb0, z2_0, e2_0)
    zq0 = _sc_gather(emb0p, i0)[:, :_DIM]

    resid = zf - zq0
    z2_1 = jnp.sum(resid ** 2, axis=1, keepdims=True)
    i1, loss1 = _quantize_stage(resid, emb1, z2_1, e2_1)
    zq1 = _sc_gather(emb1p, i1)[:, :_DIM]

    z_q = (zq0 + zq1).T.reshape(B, C, H, W)

    def to_mei(ix):
        return ix.reshape(h, 4, w, 4).transpose(0, 2, 1, 3).reshape(1, h, w, 16)

    mei = jnp.stack([to_mei(i0), to_mei(i1)], axis=-1)
    return z_q, loss0[0, 0] + loss1[0, 0], mei


# speedup vs baseline: 1.6551x; 1.1936x over previous
import functools

import jax
import jax.numpy as jnp
from jax import lax
from jax.experimental import pallas as pl
from jax.experimental.pallas import tpu as pltpu
from jax.experimental.pallas import tpu_sc as plsc

_N_EMBED = 8192
_DIM = 32
_M = 1024
_CK = 1024
_NCHUNK = _N_EMBED // _CK
_BETA = 0.25
_EPS = 1e-5


def _stage_body(z_ref, e_ref, z2_ref, e2_ref,
                idx_ref, loss_ref, counts_ref, sq_ref):
    pid = pl.program_id(0)
    nsteps = pl.num_programs(0)

    @pl.when(pid == 0)
    def _init():
        counts_ref[...] = jnp.zeros_like(counts_ref)
        sq_ref[0] = 0.0

    zt = z_ref[...]
    z2 = z2_ref[...]

    def dbody(c, carry):
        bestv, besti = carry
        off = pl.multiple_of(c * _CK, _CK)
        ech = e_ref[pl.ds(off, _CK), :]
        e2c = e2_ref[pl.ds(off, _CK)]
        mm = lax.dot_general(zt, ech, (((1,), (1,)), ((), ())),
                             preferred_element_type=jnp.float32)
        d = (z2 + e2c[None, :]) - 2.0 * mm
        lv = jnp.min(d, axis=1)
        iota = lax.broadcasted_iota(jnp.int32, (_M, _CK), 1)
        li = jnp.min(jnp.where(d == lv[:, None], iota, _CK), axis=1) + off
        upd = lv < bestv
        return jnp.where(upd, lv, bestv), jnp.where(upd, li, besti)

    bestv0 = jnp.full((_M,), jnp.inf, dtype=jnp.float32)
    besti0 = jnp.zeros((_M,), dtype=jnp.int32)
    bestv, idx = lax.fori_loop(0, _NCHUNK, dbody, (bestv0, besti0))
    idx_ref[...] = idx
    sq_ref[0] += jnp.sum(jnp.maximum(bestv, 0.0))

    def cbody(c, _):
        off = pl.multiple_of(c * _CK, _CK)
        iota = lax.broadcasted_iota(jnp.int32, (_M, _CK), 1) + off
        oh = (idx[:, None] == iota).astype(jnp.float32)
        counts_ref[pl.ds(c, 1), :] += jnp.sum(oh, axis=0)[None, :]
        return 0

    lax.fori_loop(0, _NCHUNK, cbody, 0)

    @pl.when(pid == nsteps - 1)
    def _fin():
        numel = jnp.asarray(nsteps * _M * _DIM, dtype=jnp.float32)
        counts = counts_ref[...]
        tot = jnp.sum(counts)
        probs = (counts + _EPS) / (tot + _EPS * _N_EMBED)
        ent = -jnp.sum(probs * jnp.log(probs + _EPS))
        usage = jnp.log(jnp.float32(_N_EMBED)) - ent
        loss = _BETA * sq_ref[0] / numel + 0.01 * usage
        loss_ref[...] = jnp.broadcast_to(loss, (1, 1))


def _quantize_stage(zf, emb, z2, e2):
    n_pix = zf.shape[0]
    grid = n_pix // _M
    return pl.pallas_call(
        _stage_body,
        grid=(grid,),
        in_specs=[
            pl.BlockSpec((_M, _DIM), lambda i: (i, 0)),
            pl.BlockSpec((_N_EMBED, _DIM), lambda i: (0, 0)),
            pl.BlockSpec((_M, 1), lambda i: (i, 0)),
            pl.BlockSpec((_N_EMBED,), lambda i: (0,)),
        ],
        out_specs=[
            pl.BlockSpec((_M,), lambda i: (i,)),
            pl.BlockSpec((1, 1), lambda i: (0, 0)),
        ],
        out_shape=[
            jax.ShapeDtypeStruct((n_pix,), jnp.int32),
            jax.ShapeDtypeStruct((1, 1), jnp.float32),
        ],
        scratch_shapes=[
            pltpu.VMEM((_NCHUNK, _CK), jnp.float32),
            pltpu.SMEM((1,), jnp.float32),
        ],
    )(zf, emb, z2, e2)


_GW = 128


def _sc_gather(table128, idx):
    n = idx.shape[0]
    info = plsc.get_sparse_core_info()
    nw = info.num_cores * info.num_subcores
    b_per_w = n // nw
    nchunk = 2
    bc = b_per_w // nchunk
    mesh = plsc.VectorSubcoreMesh(core_axis_name="c", subcore_axis_name="s")

    @functools.partial(
        pl.kernel, mesh=mesh,
        out_type=jax.ShapeDtypeStruct((n, _GW), jnp.float32),
        scratch_types=[
            pltpu.VMEM((bc,), jnp.int32),
            pltpu.VMEM((bc, _GW), jnp.float32),
            pltpu.SemaphoreType.DMA,
        ],
    )
    def k(table_hbm, idx_hbm, out_hbm, idx_v, rows_v, sem):
        wid = lax.axis_index("s") * info.num_cores + lax.axis_index("c")
        for j in range(nchunk):
            base = wid * b_per_w + j * bc
            pltpu.sync_copy(idx_hbm.at[pl.ds(base, bc)], idx_v)
            pltpu.async_copy(table_hbm.at[idx_v], rows_v, sem).wait()
            pltpu.sync_copy(rows_v, out_hbm.at[pl.ds(base, bc)])

    return k(table128, idx)


def kernel(z, emb0, emb1):
    B, C, H, W = z.shape
    assert B == 1 and C == _DIM
    h, w = H // 4, W // 4
    zf = z.reshape(C, H * W).T

    e2_0 = jnp.sum(emb0 ** 2, axis=1)
    e2_1 = jnp.sum(emb1 ** 2, axis=1)
    pad = ((0, 0), (0, _GW - _DIM))
    emb0p = jnp.pad(emb0, pad)
    emb1p = jnp.pad(emb1, pad)

    z2_0 = jnp.sum(zf ** 2, axis=1, keepdims=True)
    i0, loss0 = _quantize_stage(zf, emb0, z2_0, e2_0)
    zq0 = _sc_gather(emb0p, i0)[:, :_DIM]

    resid = zf - zq0
    z2_1 = jnp.sum(resid ** 2, axis=1, keepdims=True)
    i1, loss1 = _quantize_stage(resid, emb1, z2_1, e2_1)
    zq1 = _sc_gather(emb1p, i1)[:, :_DIM]

    z_q = (zq0 + zq1).T.reshape(B, C, H, W)

    def to_mei(ix):
        return ix.reshape(h, 4, w, 4).transpose(0, 2, 1, 3).reshape(1, h, w, 16)

    mei = jnp.stack([to_mei(i0), to_mei(i1)], axis=-1)
    return z_q, loss0[0, 0] + loss1[0, 0], mei


# speedup vs baseline: 1.8679x; 1.1286x over previous
import functools

import jax
import jax.numpy as jnp
from jax import lax
from jax.experimental import pallas as pl
from jax.experimental.pallas import tpu as pltpu
from jax.experimental.pallas import tpu_sc as plsc

_N_EMBED = 8192
_DIM = 32
_M = 1024
_CK = 2048
_NCHUNK = _N_EMBED // _CK
_BETA = 0.25
_EPS = 1e-5


def _stage_body(z_ref, e_ref, z2_ref, e2_ref,
                idx_ref, loss_ref, counts_ref, sq_ref):
    pid = pl.program_id(0)
    nsteps = pl.num_programs(0)

    @pl.when(pid == 0)
    def _init():
        counts_ref[...] = jnp.zeros_like(counts_ref)
        sq_ref[0] = 0.0

    zt = z_ref[...]
    z2 = z2_ref[...]

    def dbody(c, carry):
        bestv, besti = carry
        off = pl.multiple_of(c * _CK, _CK)
        ech = e_ref[pl.ds(off, _CK), :]
        e2c = e2_ref[pl.ds(off, _CK)]
        mm = lax.dot_general(zt, ech, (((1,), (1,)), ((), ())),
                             preferred_element_type=jnp.float32)
        d = (z2 + e2c[None, :]) - 2.0 * mm
        lv = jnp.min(d, axis=1)
        iota = lax.broadcasted_iota(jnp.int32, (_M, _CK), 1)
        li = jnp.min(jnp.where(d == lv[:, None], iota, _CK), axis=1) + off
        upd = lv < bestv
        return jnp.where(upd, lv, bestv), jnp.where(upd, li, besti)

    bestv0 = jnp.full((_M,), jnp.inf, dtype=jnp.float32)
    besti0 = jnp.zeros((_M,), dtype=jnp.int32)
    bestv, idx = lax.fori_loop(0, _NCHUNK, dbody, (bestv0, besti0))
    idx_ref[...] = idx
    sq_ref[0] += jnp.sum(jnp.maximum(bestv, 0.0))

    def cbody(c, _):
        off = pl.multiple_of(c * _CK, _CK)
        iota = lax.broadcasted_iota(jnp.int32, (_M, _CK), 1) + off
        oh = (idx[:, None] == iota).astype(jnp.float32)
        counts_ref[pl.ds(c, 1), :] += jnp.sum(oh, axis=0)[None, :]
        return 0

    lax.fori_loop(0, _NCHUNK, cbody, 0)

    @pl.when(pid == nsteps - 1)
    def _fin():
        numel = jnp.asarray(nsteps * _M * _DIM, dtype=jnp.float32)
        counts = counts_ref[...]
        tot = jnp.sum(counts)
        probs = (counts + _EPS) / (tot + _EPS * _N_EMBED)
        ent = -jnp.sum(probs * jnp.log(probs + _EPS))
        usage = jnp.log(jnp.float32(_N_EMBED)) - ent
        loss = _BETA * sq_ref[0] / numel + 0.01 * usage
        loss_ref[...] = jnp.broadcast_to(loss, (1, 1))


def _quantize_stage(zf, emb, z2, e2):
    n_pix = zf.shape[0]
    grid = n_pix // _M
    return pl.pallas_call(
        _stage_body,
        grid=(grid,),
        in_specs=[
            pl.BlockSpec((_M, _DIM), lambda i: (i, 0)),
            pl.BlockSpec((_N_EMBED, _DIM), lambda i: (0, 0)),
            pl.BlockSpec((_M, 1), lambda i: (i, 0)),
            pl.BlockSpec((_N_EMBED,), lambda i: (0,)),
        ],
        out_specs=[
            pl.BlockSpec((_M,), lambda i: (i,)),
            pl.BlockSpec((1, 1), lambda i: (0, 0)),
        ],
        out_shape=[
            jax.ShapeDtypeStruct((n_pix,), jnp.int32),
            jax.ShapeDtypeStruct((1, 1), jnp.float32),
        ],
        scratch_shapes=[
            pltpu.VMEM((_NCHUNK, _CK), jnp.float32),
            pltpu.SMEM((1,), jnp.float32),
        ],
    )(zf, emb, z2, e2)


_GW = 128


def _sc_gather(table128, idx):
    n = idx.shape[0]
    info = plsc.get_sparse_core_info()
    nw = info.num_cores * info.num_subcores
    b_per_w = n // nw
    nchunk = 2
    bc = b_per_w // nchunk
    mesh = plsc.VectorSubcoreMesh(core_axis_name="c", subcore_axis_name="s")

    @functools.partial(
        pl.kernel, mesh=mesh,
        out_type=jax.ShapeDtypeStruct((n, _GW), jnp.float32),
        scratch_types=[
            pltpu.VMEM((bc,), jnp.int32),
            pltpu.VMEM((bc, _GW), jnp.float32),
            pltpu.SemaphoreType.DMA,
        ],
    )
    def k(table_hbm, idx_hbm, out_hbm, idx_v, rows_v, sem):
        wid = lax.axis_index("s") * info.num_cores + lax.axis_index("c")
        for j in range(nchunk):
            base = wid * b_per_w + j * bc
            pltpu.sync_copy(idx_hbm.at[pl.ds(base, bc)], idx_v)
            pltpu.async_copy(table_hbm.at[idx_v], rows_v, sem).wait()
            pltpu.sync_copy(rows_v, out_hbm.at[pl.ds(base, bc)])

    return k(table128, idx)


def kernel(z, emb0, emb1):
    B, C, H, W = z.shape
    assert B == 1 and C == _DIM
    h, w = H // 4, W // 4
    zf = z.reshape(C, H * W).T

    e2_0 = jnp.sum(emb0 ** 2, axis=1)
    e2_1 = jnp.sum(emb1 ** 2, axis=1)
    pad = ((0, 0), (0, _GW - _DIM))
    emb0p = jnp.pad(emb0, pad)
    emb1p = jnp.pad(emb1, pad)

    z2_0 = jnp.sum(zf ** 2, axis=1, keepdims=True)
    i0, loss0 = _quantize_stage(zf, emb0, z2_0, e2_0)
    zq0 = _sc_gather(emb0p, i0)[:, :_DIM]

    resid = zf - zq0
    z2_1 = jnp.sum(resid ** 2, axis=1, keepdims=True)
    i1, loss1 = _quantize_stage(resid, emb1, z2_1, e2_1)
    zq1 = _sc_gather(emb1p, i1)[:, :_DIM]

    z_q = (zq0 + zq1).T.reshape(B, C, H, W)

    def to_mei(ix):
        return ix.reshape(h, 4, w, 4).transpose(0, 2, 1, 3).reshape(1, h, w, 16)

    mei = jnp.stack([to_mei(i0), to_mei(i1)], axis=-1)
    return z_q, loss0[0, 0] + loss1[0, 0], mei


# speedup vs baseline: 2.0137x; 1.0781x over previous
import functools

import jax
import jax.numpy as jnp
from jax import lax
from jax.experimental import pallas as pl
from jax.experimental.pallas import tpu as pltpu
from jax.experimental.pallas import tpu_sc as plsc

_N_EMBED = 8192
_DIM = 32
_M = 1024
_CK = 4096
_NCHUNK = _N_EMBED // _CK
_BETA = 0.25
_EPS = 1e-5


def _stage_body(z_ref, e_ref, z2_ref, e2_ref,
                idx_ref, loss_ref, counts_ref, sq_ref):
    pid = pl.program_id(0)
    nsteps = pl.num_programs(0)

    @pl.when(pid == 0)
    def _init():
        counts_ref[...] = jnp.zeros_like(counts_ref)
        sq_ref[0] = 0.0

    zt = z_ref[...]
    z2 = z2_ref[...]

    def dbody(c, carry):
        bestv, besti = carry
        off = pl.multiple_of(c * _CK, _CK)
        ech = e_ref[pl.ds(off, _CK), :]
        e2c = e2_ref[pl.ds(off, _CK)]
        mm = lax.dot_general(zt, ech, (((1,), (1,)), ((), ())),
                             preferred_element_type=jnp.float32)
        d = (z2 + e2c[None, :]) - 2.0 * mm
        lv = jnp.min(d, axis=1)
        iota = lax.broadcasted_iota(jnp.int32, (_M, _CK), 1)
        li = jnp.min(jnp.where(d == lv[:, None], iota, _CK), axis=1) + off
        upd = lv < bestv
        return jnp.where(upd, lv, bestv), jnp.where(upd, li, besti)

    bestv0 = jnp.full((_M,), jnp.inf, dtype=jnp.float32)
    besti0 = jnp.zeros((_M,), dtype=jnp.int32)
    bestv, idx = lax.fori_loop(0, _NCHUNK, dbody, (bestv0, besti0))
    idx_ref[...] = idx
    sq_ref[0] += jnp.sum(jnp.maximum(bestv, 0.0))

    def cbody(c, _):
        off = pl.multiple_of(c * _CK, _CK)
        iota = lax.broadcasted_iota(jnp.int32, (_M, _CK), 1) + off
        oh = (idx[:, None] == iota).astype(jnp.float32)
        counts_ref[pl.ds(c, 1), :] += jnp.sum(oh, axis=0)[None, :]
        return 0

    lax.fori_loop(0, _NCHUNK, cbody, 0)

    @pl.when(pid == nsteps - 1)
    def _fin():
        numel = jnp.asarray(nsteps * _M * _DIM, dtype=jnp.float32)
        counts = counts_ref[...]
        tot = jnp.sum(counts)
        probs = (counts + _EPS) / (tot + _EPS * _N_EMBED)
        ent = -jnp.sum(probs * jnp.log(probs + _EPS))
        usage = jnp.log(jnp.float32(_N_EMBED)) - ent
        loss = _BETA * sq_ref[0] / numel + 0.01 * usage
        loss_ref[...] = jnp.broadcast_to(loss, (1, 1))


def _quantize_stage(zf, emb, z2, e2):
    n_pix = zf.shape[0]
    grid = n_pix // _M
    return pl.pallas_call(
        _stage_body,
        grid=(grid,),
        in_specs=[
            pl.BlockSpec((_M, _DIM), lambda i: (i, 0)),
            pl.BlockSpec((_N_EMBED, _DIM), lambda i: (0, 0)),
            pl.BlockSpec((_M, 1), lambda i: (i, 0)),
            pl.BlockSpec((_N_EMBED,), lambda i: (0,)),
        ],
        out_specs=[
            pl.BlockSpec((_M,), lambda i: (i,)),
            pl.BlockSpec((1, 1), lambda i: (0, 0)),
        ],
        out_shape=[
            jax.ShapeDtypeStruct((n_pix,), jnp.int32),
            jax.ShapeDtypeStruct((1, 1), jnp.float32),
        ],
        scratch_shapes=[
            pltpu.VMEM((_NCHUNK, _CK), jnp.float32),
            pltpu.SMEM((1,), jnp.float32),
        ],
    )(zf, emb, z2, e2)


_GW = 128


def _sc_gather(table128, idx):
    n = idx.shape[0]
    info = plsc.get_sparse_core_info()
    nw = info.num_cores * info.num_subcores
    b_per_w = n // nw
    nchunk = 2
    bc = b_per_w // nchunk
    mesh = plsc.VectorSubcoreMesh(core_axis_name="c", subcore_axis_name="s")

    @functools.partial(
        pl.kernel, mesh=mesh,
        out_type=jax.ShapeDtypeStruct((n, _GW), jnp.float32),
        scratch_types=[
            pltpu.VMEM((bc,), jnp.int32),
            pltpu.VMEM((bc, _GW), jnp.float32),
            pltpu.SemaphoreType.DMA,
        ],
    )
    def k(table_hbm, idx_hbm, out_hbm, idx_v, rows_v, sem):
        wid = lax.axis_index("s") * info.num_cores + lax.axis_index("c")
        for j in range(nchunk):
            base = wid * b_per_w + j * bc
            pltpu.sync_copy(idx_hbm.at[pl.ds(base, bc)], idx_v)
            pltpu.async_copy(table_hbm.at[idx_v], rows_v, sem).wait()
            pltpu.sync_copy(rows_v, out_hbm.at[pl.ds(base, bc)])

    return k(table128, idx)


def kernel(z, emb0, emb1):
    B, C, H, W = z.shape
    assert B == 1 and C == _DIM
    h, w = H // 4, W // 4
    zf = z.reshape(C, H * W).T

    e2_0 = jnp.sum(emb0 ** 2, axis=1)
    e2_1 = jnp.sum(emb1 ** 2, axis=1)
    pad = ((0, 0), (0, _GW - _DIM))
    emb0p = jnp.pad(emb0, pad)
    emb1p = jnp.pad(emb1, pad)

    z2_0 = jnp.sum(zf ** 2, axis=1, keepdims=True)
    i0, loss0 = _quantize_stage(zf, emb0, z2_0, e2_0)
    zq0 = _sc_gather(emb0p, i0)[:, :_DIM]

    resid = zf - zq0
    z2_1 = jnp.sum(resid ** 2, axis=1, keepdims=True)
    i1, loss1 = _quantize_stage(resid, emb1, z2_1, e2_1)
    zq1 = _sc_gather(emb1p, i1)[:, :_DIM]

    z_q = (zq0 + zq1).T.reshape(B, C, H, W)

    def to_mei(ix):
        return ix.reshape(h, 4, w, 4).transpose(0, 2, 1, 3).reshape(1, h, w, 16)

    mei = jnp.stack([to_mei(i0), to_mei(i1)], axis=-1)
    return z_q, loss0[0, 0] + loss1[0, 0], mei
